# baseline (speedup 1.0000x reference)
import functools

import jax
import jax.numpy as jnp
from jax import lax
from jax.experimental import pallas as pl
from jax.experimental.pallas import tpu as pltpu
from jax.experimental.pallas import tpu_sc as plsc

NUM_EMBEDDINGS = 100000
D = 64
SEQ = 200
B = 4096
NC = 2
NS = 16
NW = NC * NS
BBLK = B // NW
LANES = 16
NB = 5
LOOK = 4
LGRP = 8
NGRP = SEQ // LGRP
TILES = D // 8
PADW = 137


def _emb_body(x_hbm, table_hbm, pos_hbm, out_hbm, pos_v, cidx_v, idx_v,
              r0, r1, r2, r3, r4, o0, o1, o2, o3, o4,
              isem, gs0, gs1, gs2, gs3, gs4, ss0, ss1, ss2, ss3, ss4):
    rows = (r0, r1, r2, r3, r4)
    outs = (o0, o1, o2, o3, o4)
    gsem = (gs0, gs1, gs2, gs3, gs4)
    ssem = (ss0, ss1, ss2, ss3, ss4)

    wid = lax.axis_index("s") * NC + lax.axis_index("c")
    pltpu.sync_copy(pos_hbm, pos_v)

    for c in range(D // LANES):
        cidx_v[c, :] = lax.iota(jnp.int32, LANES) + (LANES * c)

    def issue_idx(g, s):
        pltpu.async_copy(x_hbm.at[g, wid], idx_v.at[s], isem)

    def wait_idx(g, s):
        pltpu.make_async_copy(x_hbm.at[g, wid], idx_v.at[s], isem).wait()

    def issue_gather(l, b):
        pltpu.async_copy(table_hbm.at[idx_v.at[(l // LGRP) % 2, l % LGRP]],
                         rows[b], gsem[b])

    def wait_gather(l, b):
        pltpu.make_async_copy(
            table_hbm.at[idx_v.at[(l // LGRP) % 2, l % LGRP]],
            rows[b], gsem[b]).wait()

    def issue_store(l, b):
        for t in range(TILES):
            pltpu.async_copy(outs[b].at[pl.ds(t * 8, 8), pl.ds(0, BBLK)],
                             out_hbm.at[l, t, wid], ssem[b])

    def wait_store(l, b):
        for t in range(TILES):
            pltpu.make_async_copy(outs[b].at[pl.ds(t * 8, 8), pl.ds(0, BBLK)],
                                  out_hbm.at[l, t, wid], ssem[b]).wait()

    def compute(l, b):
        pvecs = [pos_v[l, pl.ds(c * LANES, LANES)] for c in range(D // LANES)]
        cvecs = [cidx_v[c, :] for c in range(D // LANES)]

        @plsc.parallel_loop(0, BBLK, 1, unroll=8)
        def bb_body(bb):
            sb = jnp.broadcast_to(bb, (LANES,))
            for c in range(D // LANES):
                val = rows[b][bb, pl.ds(c * LANES, LANES)] + pvecs[c]
                plsc.store_scatter(outs[b], [cvecs[c], sb], val)

    def stage(l, b):
        wait_gather(l, b)

        @pl.when(jnp.logical_and((l & (LGRP - 1)) == 0,
                                 jnp.logical_and(l > 0, l + LGRP < SEQ)))
        def _():
            g1 = (l >> 3) + 1
            issue_idx(g1, g1 & 1)

        @pl.when(l >= NB)
        def _():
            wait_store(l - NB, b)

        compute(l, b)
        issue_store(l, b)

        @pl.when(jnp.logical_and(l + LOOK < SEQ,
                                 ((l + LOOK) & (LGRP - 1)) == 0))
        def _():
            wait_idx((l + LOOK) >> 3, ((l + LOOK) >> 3) & 1)

        @pl.when(l + LOOK < SEQ)
        def _():
            issue_gather(l + LOOK, (b + LOOK) % NB)

    issue_idx(0, 0)
    wait_idx(0, 0)
    issue_idx(1, 1)
    for j in range(LOOK):
        issue_gather(j, j)

    def quad_body(t, carry):
        for j in range(NB):
            stage(t * NB + j, j)
        return carry
    lax.fori_loop(0, SEQ // NB, quad_body, 0)

    for j in range(NB):
        l = SEQ - NB + j
        wait_store(l, l % NB)


@jax.jit
def _emb(x_r, table, pos_enc):
    mesh = plsc.VectorSubcoreMesh(core_axis_name="c", subcore_axis_name="s")
    f = functools.partial(
        pl.kernel,
        mesh=mesh,
        out_type=jax.ShapeDtypeStruct((SEQ, TILES, NW, 8, BBLK), jnp.float32),
        scratch_types=(
            [pltpu.VMEM((SEQ, D), jnp.float32),
             pltpu.VMEM((D // LANES, LANES), jnp.int32),
             pltpu.VMEM((2, LGRP, BBLK), jnp.int32)]
            + [pltpu.VMEM((BBLK, D), jnp.float32) for _ in range(NB)]
            + [pltpu.VMEM((D, PADW), jnp.float32) for _ in range(NB)]
            + [pltpu.SemaphoreType.DMA] * (1 + 2 * NB)
        ),
        compiler_params=pltpu.CompilerParams(use_tc_tiling_on_sc=False,
                                             needs_layout_passes=False),
    )(_emb_body)
    return f(x_r, table, pos_enc)


def kernel(x, table, pos_enc):
    x_r = jnp.transpose(x.astype(jnp.int32).reshape(NW, BBLK, SEQ // 8, 8),
                        (2, 0, 3, 1))
    out5 = _emb(x_r, table, pos_enc)
    return out5.transpose(2, 4, 0, 1, 3).reshape(B, SEQ, D)


# speedup vs baseline: 1.2390x; 1.2390x over previous
import functools

import jax
import jax.numpy as jnp
from jax import lax
from jax.experimental import pallas as pl
from jax.experimental.pallas import tpu as pltpu
from jax.experimental.pallas import tpu_sc as plsc

NUM_EMBEDDINGS = 100000
D = 64
SEQ = 200
B = 4096
NC = 2
NS = 16
NW = NC * NS
BBLK = B // NW
LANES = 16
NB = 5
LOOK = 4
LGRP = 8
NGRP = SEQ // LGRP
TILES = D // 8
PADW = 137


def _emb_body(x_hbm, table_hbm, pos_hbm, out_hbm, pos_v, cidx_v, idx_v,
              r0, r1, r2, r3, r4, o0, o1, o2, o3, o4,
              isem, gs0, gs1, gs2, gs3, gs4, ss0, ss1, ss2, ss3, ss4):
    rows = (r0, r1, r2, r3, r4)
    outs = (o0, o1, o2, o3, o4)
    gsem = (gs0, gs1, gs2, gs3, gs4)
    ssem = (ss0, ss1, ss2, ss3, ss4)

    wid = lax.axis_index("s") * NC + lax.axis_index("c")
    pltpu.sync_copy(pos_hbm, pos_v)

    for c in range(D // LANES):
        cidx_v[c, :] = lax.iota(jnp.int32, LANES) + (LANES * c)

    def issue_idx(g, s):
        pltpu.async_copy(x_hbm.at[g, wid], idx_v.at[s], isem)

    def wait_idx(g, s):
        pltpu.make_async_copy(x_hbm.at[g, wid], idx_v.at[s], isem).wait()

    def issue_gather(l, b):
        pltpu.async_copy(table_hbm.at[idx_v.at[(l // LGRP) % 2, l % LGRP]],
                         rows[b], gsem[b])

    def wait_gather(l, b):
        pltpu.make_async_copy(
            table_hbm.at[idx_v.at[(l // LGRP) % 2, l % LGRP]],
            rows[b], gsem[b]).wait()

    def issue_store(l, b):
        for t in range(TILES):
            pltpu.async_copy(outs[b].at[pl.ds(t * 8, 8), pl.ds(0, BBLK)],
                             out_hbm.at[l, t, wid], ssem[b])

    def wait_store(l, b):
        for t in range(TILES):
            pltpu.make_async_copy(outs[b].at[pl.ds(t * 8, 8), pl.ds(0, BBLK)],
                                  out_hbm.at[l, t, wid], ssem[b]).wait()

    def compute(l, b):
        pvecs = [pos_v[l, pl.ds(c * LANES, LANES)] for c in range(D // LANES)]
        cvecs = [cidx_v[c, :] for c in range(D // LANES)]

        @plsc.parallel_loop(0, BBLK, 1, unroll=2)
        def bb_body(bb):
            sb = jnp.broadcast_to(bb, (LANES,))
            for c in range(D // LANES):
                val = rows[b][bb, pl.ds(c * LANES, LANES)] + pvecs[c]
                plsc.store_scatter(outs[b], [cvecs[c], sb], val)

    def stage(l, b):
        wait_gather(l, b)

        @pl.when(jnp.logical_and((l & (LGRP - 1)) == 0,
                                 jnp.logical_and(l > 0, l + LGRP < SEQ)))
        def _():
            g1 = (l >> 3) + 1
            issue_idx(g1, g1 & 1)

        @pl.when(l >= NB)
        def _():
            wait_store(l - NB, b)

        compute(l, b)
        issue_store(l, b)

        @pl.when(jnp.logical_and(l + LOOK < SEQ,
                                 ((l + LOOK) & (LGRP - 1)) == 0))
        def _():
            wait_idx((l + LOOK) >> 3, ((l + LOOK) >> 3) & 1)

        @pl.when(l + LOOK < SEQ)
        def _():
            issue_gather(l + LOOK, (b + LOOK) % NB)

    issue_idx(0, 0)
    wait_idx(0, 0)
    issue_idx(1, 1)
    for j in range(LOOK):
        issue_gather(j, j)

    def quad_body(t, carry):
        for j in range(NB):
            stage(t * NB + j, j)
        return carry
    lax.fori_loop(0, SEQ // NB, quad_body, 0)

    for j in range(NB):
        l = SEQ - NB + j
        wait_store(l, l % NB)


@jax.jit
def _emb(x_r, table, pos_enc):
    mesh = plsc.VectorSubcoreMesh(core_axis_name="c", subcore_axis_name="s")
    f = functools.partial(
        pl.kernel,
        mesh=mesh,
        out_type=jax.ShapeDtypeStruct((SEQ, TILES, NW, 8, BBLK), jnp.float32),
        scratch_types=(
            [pltpu.VMEM((SEQ, D), jnp.float32),
             pltpu.VMEM((D // LANES, LANES), jnp.int32),
             pltpu.VMEM((2, LGRP, BBLK), jnp.int32)]
            + [pltpu.VMEM((BBLK, D), jnp.float32) for _ in range(NB)]
            + [pltpu.VMEM((D, PADW), jnp.float32) for _ in range(NB)]
            + [pltpu.SemaphoreType.DMA] * (1 + 2 * NB)
        ),
        compiler_params=pltpu.CompilerParams(use_tc_tiling_on_sc=False,
                                             needs_layout_passes=False),
    )(_emb_body)
    return f(x_r, table, pos_enc)


def kernel(x, table, pos_enc):
    x_r = jnp.transpose(x.astype(jnp.int32).reshape(NW, BBLK, SEQ // 8, 8),
                        (2, 0, 3, 1))
    out5 = _emb(x_r, table, pos_enc)
    return out5.transpose(2, 4, 0, 1, 3).reshape(B, SEQ, D)


# speedup vs baseline: 1.2469x; 1.0063x over previous
import functools

import jax
import jax.numpy as jnp
from jax import lax
from jax.experimental import pallas as pl
from jax.experimental.pallas import tpu as pltpu
from jax.experimental.pallas import tpu_sc as plsc

NUM_EMBEDDINGS = 100000
D = 64
SEQ = 200
B = 4096
NC = 2
NS = 16
NW = NC * NS
BBLK = B // NW
LANES = 16
NB = 5
LOOK = 4
LGRP = 8
NGRP = SEQ // LGRP
TILES = D // 8
PADW = 137


def _emb_body(x_hbm, table_hbm, pos_hbm, out_hbm, pos_v, cidx_v, idx_v,
              r0, r1, r2, r3, r4, o0, o1, o2, o3, o4,
              isem, gs0, gs1, gs2, gs3, gs4, ss0, ss1, ss2, ss3, ss4):
    rows = (r0, r1, r2, r3, r4)
    outs = (o0, o1, o2, o3, o4)
    gsem = (gs0, gs1, gs2, gs3, gs4)
    ssem = (ss0, ss1, ss2, ss3, ss4)

    wid = lax.axis_index("s") * NC + lax.axis_index("c")
    pltpu.sync_copy(pos_hbm, pos_v)

    for c in range(D // LANES):
        d = lax.iota(jnp.int32, LANES) + (LANES * c)
        cidx_v[0, c, :] = d >> 3
        cidx_v[1, c, :] = d & 7

    def issue_idx(g, s):
        pltpu.async_copy(x_hbm.at[g, wid], idx_v.at[s], isem)

    def wait_idx(g, s):
        pltpu.make_async_copy(x_hbm.at[g, wid], idx_v.at[s], isem).wait()

    def issue_gather(l, b):
        pltpu.async_copy(table_hbm.at[idx_v.at[(l // LGRP) % 2, l % LGRP]],
                         rows[b], gsem[b])

    def wait_gather(l, b):
        pltpu.make_async_copy(
            table_hbm.at[idx_v.at[(l // LGRP) % 2, l % LGRP]],
            rows[b], gsem[b]).wait()

    def issue_store(l, b):
        pltpu.async_copy(outs[b].at[:, :, pl.ds(0, BBLK)],
                         out_hbm.at[l, :, wid], ssem[b])

    def wait_store(l, b):
        pltpu.make_async_copy(outs[b].at[:, :, pl.ds(0, BBLK)],
                              out_hbm.at[l, :, wid], ssem[b]).wait()

    def compute(l, b):
        pvecs = [pos_v[l, pl.ds(c * LANES, LANES)] for c in range(D // LANES)]
        tds = [cidx_v[0, c, :] for c in range(D // LANES)]
        dds = [cidx_v[1, c, :] for c in range(D // LANES)]

        @plsc.parallel_loop(0, BBLK, 1, unroll=4)
        def bb_body(bb):
            sb = jnp.broadcast_to(bb, (LANES,))
            for c in range(D // LANES):
                val = rows[b][bb, pl.ds(c * LANES, LANES)] + pvecs[c]
                plsc.store_scatter(outs[b], [tds[c], dds[c], sb], val)

    def stage(l, b):
        wait_gather(l, b)

        @pl.when(jnp.logical_and((l & (LGRP - 1)) == 0,
                                 jnp.logical_and(l > 0, l + LGRP < SEQ)))
        def _():
            g1 = (l >> 3) + 1
            issue_idx(g1, g1 & 1)

        @pl.when(l >= NB)
        def _():
            wait_store(l - NB, b)

        compute(l, b)
        issue_store(l, b)

        @pl.when(jnp.logical_and(l + LOOK < SEQ,
                                 ((l + LOOK) & (LGRP - 1)) == 0))
        def _():
            wait_idx((l + LOOK) >> 3, ((l + LOOK) >> 3) & 1)

        @pl.when(l + LOOK < SEQ)
        def _():
            issue_gather(l + LOOK, (b + LOOK) % NB)

    issue_idx(0, 0)
    wait_idx(0, 0)
    issue_idx(1, 1)
    for j in range(LOOK):
        issue_gather(j, j)

    def quad_body(t, carry):
        for j in range(NB):
            stage(t * NB + j, j)
        return carry
    lax.fori_loop(0, SEQ // NB, quad_body, 0)

    for j in range(NB):
        l = SEQ - NB + j
        wait_store(l, l % NB)


@jax.jit
def _emb(x_r, table, pos_enc):
    mesh = plsc.VectorSubcoreMesh(core_axis_name="c", subcore_axis_name="s")
    f = functools.partial(
        pl.kernel,
        mesh=mesh,
        out_type=jax.ShapeDtypeStruct((SEQ, TILES, NW, 8, BBLK), jnp.float32),
        scratch_types=(
            [pltpu.VMEM((SEQ, D), jnp.float32),
             pltpu.VMEM((2, D // LANES, LANES), jnp.int32),
             pltpu.VMEM((2, LGRP, BBLK), jnp.int32)]
            + [pltpu.VMEM((BBLK, D), jnp.float32) for _ in range(NB)]
            + [pltpu.VMEM((TILES, 8, PADW), jnp.float32) for _ in range(NB)]
            + [pltpu.SemaphoreType.DMA] * (1 + 2 * NB)
        ),
        compiler_params=pltpu.CompilerParams(use_tc_tiling_on_sc=False,
                                             needs_layout_passes=False),
    )(_emb_body)
    return f(x_r, table, pos_enc)


def kernel(x, table, pos_enc):
    x_r = jnp.transpose(x.astype(jnp.int32).reshape(NW, BBLK, SEQ // 8, 8),
                        (2, 0, 3, 1))
    out5 = _emb(x_r, table, pos_enc)
    return out5.transpose(2, 4, 0, 1, 3).reshape(B, SEQ, D)
